# baseline (speedup 1.0000x reference)
import functools

import jax
import jax.numpy as jnp
from jax import lax
from jax.experimental import pallas as pl
from jax.experimental.pallas import tpu as pltpu
from jax.experimental.pallas import tpu_sc as plsc

N = 10000
NPAD = 10240
E = 320000
P = 100000
DP = 128
DH = DP // 2
PPAD = 102400
NC, NS = 2, 16
NW = NC * NS
EPS = E // NS
RPT = NPAD // NS
PPW = PPAD // NW
KE = 80
KP = 80


def _linear(vf, Wp, bp):
    BM = 400

    def body(x_ref, w_ref, b_ref, o_ref):
        res = (
            jnp.dot(x_ref[...], w_ref[...], preferred_element_type=jnp.float32)
            + b_ref[...]
        )
        o_ref[0] = res[:, :DH]
        o_ref[1] = res[:, DH:]

    return pl.pallas_call(
        body,
        grid=(N // BM,),
        in_specs=[
            pl.BlockSpec((BM, DP), lambda i: (i, 0)),
            pl.BlockSpec((DP, DP), lambda i: (0, 0)),
            pl.BlockSpec((1, DP), lambda i: (0, 0)),
        ],
        out_specs=pl.BlockSpec((NC, BM, DH), lambda i: (0, i, 0)),
        out_shape=jax.ShapeDtypeStruct((NC, N, DH), jnp.float32),
    )(vf, Wp, bp)


def _edge_agg(h2, src, dst):
    mesh = plsc.VectorSubcoreMesh(core_axis_name="c", subcore_axis_name="s")
    nchunk = EPS // KE

    @functools.partial(
        pl.kernel,
        out_type=jax.ShapeDtypeStruct((NC, NPAD, DH), jnp.float32),
        mesh=mesh,
        compiler_params=pltpu.CompilerParams(use_tc_tiling_on_sc=False),
        scratch_types=[
            pltpu.VMEM((2, KE), jnp.int32),
            pltpu.VMEM((2, KE), jnp.int32),
            pltpu.VMEM((2, KE, DH), jnp.float32),
            pltpu.VMEM((RPT, DH), jnp.float32),
            pltpu.VMEM_SHARED((NPAD, DH), jnp.float32),
            pltpu.SemaphoreType.DMA,
            pltpu.SemaphoreType.DMA,
        ],
    )
    def k(h_hbm, src_hbm, dst_hbm, out_hbm, sidx, didx, rows, buf, acc,
          sem0, sem1):
        cid = lax.axis_index("c")
        sid = lax.axis_index("s")
        gsems = (sem0, sem1)

        @pl.loop(0, RPT, step=8)
        def _(r):
            for dr in range(8):
                for l in range(DH // 16):
                    buf[r + dr, pl.ds(l * 16, 16)] = jnp.zeros((16,), jnp.float32)

        pltpu.sync_copy(buf, acc.at[pl.ds(sid * RPT, RPT)])
        plsc.subcore_barrier()

        def fetch_and_fire(chunk, b):
            base = sid * EPS + chunk * KE
            pltpu.sync_copy(src_hbm.at[pl.ds(base, KE)], sidx.at[b])
            pltpu.sync_copy(dst_hbm.at[pl.ds(base, KE)], didx.at[b])
            pltpu.async_copy(h_hbm.at[cid].at[sidx.at[b]], rows.at[b], gsems[b])

        fetch_and_fire(0, 0)
        fetch_and_fire(1, 1)

        @pl.loop(0, nchunk, step=2)
        def _(it):
            for b in range(2):
                pltpu.make_async_copy(
                    h_hbm.at[cid].at[sidx.at[b]], rows.at[b], gsems[b]
                ).wait()
                pltpu.sync_copy(rows.at[b], acc.at[didx.at[b]], add=True)

                @pl.when(it + b + 2 < nchunk)
                def _():
                    fetch_and_fire(it + b + 2, b)

        plsc.subcore_barrier()
        pltpu.sync_copy(acc.at[pl.ds(sid * RPT, RPT)], buf)
        pltpu.sync_copy(buf, out_hbm.at[cid, pl.ds(sid * RPT, RPT)])

    return k(h2, src, dst)


def _finalize_gf(partials):
    BM = 512

    def body(p_ref, o_ref):
        x0 = p_ref[0]
        x1 = p_ref[1]
        li = lax.broadcasted_iota(jnp.int32, (BM, DH), 1)
        deg = jnp.sum(jnp.where(li == 100 - DH, x1, 0.0), axis=1, keepdims=True)
        r = 1.0 / jnp.maximum(deg, 1.0)
        o_ref[...] = jnp.concatenate(
            [jnp.maximum(x0 * r, 0.0), jnp.maximum(x1 * r, 0.0)], axis=1
        ).astype(jnp.bfloat16)

    return pl.pallas_call(
        body,
        grid=(NPAD // BM,),
        in_specs=[pl.BlockSpec((NC, BM, DH), lambda i: (0, i, 0))],
        out_specs=pl.BlockSpec((BM, DP), lambda i: (i, 0)),
        out_shape=jax.ShapeDtypeStruct((NPAD, DP), jnp.bfloat16),
    )(partials)


def _pair_diff(gf32, pa, pb):
    mesh = plsc.VectorSubcoreMesh(core_axis_name="c", subcore_axis_name="s")
    nchunk = PPW // KP
    assert nchunk % 2 == 0 and nchunk >= 4
    W32 = DP // 2

    @functools.partial(
        pl.kernel,
        out_type=jax.ShapeDtypeStruct((PPAD, W32), jnp.int32),
        mesh=mesh,
        compiler_params=pltpu.CompilerParams(
            use_tc_tiling_on_sc=False, needs_layout_passes=False
        ),
        scratch_types=[
            pltpu.VMEM((2, KP), jnp.int32),
            pltpu.VMEM((2, KP), jnp.int32),
            pltpu.VMEM((2, KP, W32), jnp.int32),
            pltpu.VMEM((2, KP, W32), jnp.int32),
            pltpu.SemaphoreType.DMA,
            pltpu.SemaphoreType.DMA,
            pltpu.SemaphoreType.DMA,
            pltpu.SemaphoreType.DMA,
        ],
    )
    def k(gf_hbm, pa_hbm, pb_hbm, out_hbm, ia, ib, ra, rb,
          sa0, sa1, sb0, sb1):
        cid = lax.axis_index("c")
        sid = lax.axis_index("s")
        w = cid * NS + sid
        sA = (sa0, sa1)
        sB = (sb0, sb1)

        def fetch_and_fire(chunk, b):
            base = w * PPW + chunk * KP
            pltpu.sync_copy(pa_hbm.at[pl.ds(base, KP)], ia.at[b])
            pltpu.sync_copy(pb_hbm.at[pl.ds(base, KP)], ib.at[b])
            pltpu.async_copy(gf_hbm.at[ia.at[b]], ra.at[b], sA[b])
            pltpu.async_copy(gf_hbm.at[ib.at[b]], rb.at[b], sB[b])

        fetch_and_fire(0, 0)
        fetch_and_fire(1, 1)

        @pl.loop(0, nchunk, step=2)
        def _(it):
            for b in range(2):
                pltpu.make_async_copy(
                    gf_hbm.at[ia.at[b]], ra.at[b], sA[b]
                ).wait()
                pltpu.make_async_copy(
                    gf_hbm.at[ib.at[b]], rb.at[b], sB[b]
                ).wait()

                @pl.loop(0, KP, step=4)
                def _(r):
                    for dr in range(4):
                        for l in range(W32 // 16):
                            sl = pl.ds(l * 16, 16)
                            va = plsc.bitcast(ra[b, r + dr, sl], jnp.bfloat16)
                            vb = plsc.bitcast(rb[b, r + dr, sl], jnp.bfloat16)
                            ra[b, r + dr, sl] = plsc.bitcast(
                                jnp.abs(va - vb), jnp.int32
                            )

                base = w * PPW + (it + b) * KP
                pltpu.sync_copy(ra.at[b], out_hbm.at[pl.ds(base, KP)])

                @pl.when(it + b + 2 < nchunk)
                def _():
                    fetch_and_fire(it + b + 2, b)

    return k(gf32, pa, pb)


def _head_loss(dmat, W1p, b1p, w2p, b2, tvec):
    BM = 1024
    G = PPAD // BM
    SCALE = 1.0 / P

    def body(d_ref, w1_ref, b1_ref, w2_ref, b2_ref, t_ref, o_ref):
        i = pl.program_id(0)
        d = d_ref[...]
        hdn = jnp.maximum(
            jnp.dot(d, w1_ref[...], preferred_element_type=jnp.float32)
            + b1_ref[...],
            0.0,
        )
        lg = jnp.sum(hdn * w2_ref[...], axis=1, keepdims=True) + b2_ref[...]
        t = t_ref[...]
        row = i * BM + lax.broadcasted_iota(jnp.int32, (BM, 1), 0)
        wgt = jnp.where(row < P, SCALE, 0.0)
        bce = jnp.maximum(lg, 0.0) - lg * t + jnp.log1p(jnp.exp(-jnp.abs(lg)))
        part = jnp.reshape(jnp.sum(bce * wgt), (1, 1))

        @pl.when(i == 0)
        def _():
            o_ref[...] = part

        @pl.when(i > 0)
        def _():
            o_ref[...] += part

    return pl.pallas_call(
        body,
        grid=(G,),
        in_specs=[
            pl.BlockSpec((BM, DP), lambda i: (i, 0)),
            pl.BlockSpec((DP, DP), lambda i: (0, 0)),
            pl.BlockSpec((1, DP), lambda i: (0, 0)),
            pl.BlockSpec((1, DP), lambda i: (0, 0)),
            pl.BlockSpec((1, 1), lambda i: (0, 0)),
            pl.BlockSpec((BM, 1), lambda i: (i, 0)),
        ],
        out_specs=pl.BlockSpec((1, 1), lambda i: (0, 0)),
        out_shape=jax.ShapeDtypeStruct((1, 1), jnp.float32),
    )(dmat, W1p, b1p, w2p, b2, tvec)


def kernel(vertex_features, edge_index, pairs_cells, pairs_cols, pairs_rows,
           targets_cells, targets_cols, targets_rows,
           W_gcnn, b_gcnn, W_h1, b_h1, W_h2, b_h2):
    f32 = jnp.float32
    src = edge_index[0]
    dst = edge_index[1]

    Wp = jnp.pad(W_gcnn, ((0, 0), (0, DP - 100)))
    bp = jnp.concatenate(
        [b_gcnn, jnp.ones((1,), f32), jnp.zeros((DP - 101,), f32)]
    ).reshape(1, DP)
    W1p = jnp.pad(W_h1, ((0, DP - 100), (0, DP - 50))).astype(jnp.bfloat16)
    b1p = jnp.pad(b_h1, (0, DP - 50)).reshape(1, DP)
    w2p = jnp.pad(W_h2[:, 0], (0, DP - 50)).reshape(1, DP)
    b2 = b_h2.reshape(1, 1)

    def padset(x):
        return jnp.pad(x, (0, PPAD - P))

    h2 = _linear(vertex_features, Wp, bp)
    partials = _edge_agg(h2, src, dst)
    gf = _finalize_gf(partials)
    gf32 = jax.lax.bitcast_convert_type(
        gf.reshape(NPAD, DP // 2, 2), jnp.int32
    )

    total = None
    for pairs, targets in (
        (pairs_cells, targets_cells),
        (pairs_cols, targets_cols),
        (pairs_rows, targets_rows),
    ):
        pa = padset(pairs[:, 0])
        pb = padset(pairs[:, 1])
        tvec = padset(targets.astype(f32)).reshape(PPAD, 1)
        dmat32 = _pair_diff(gf32, pa, pb)
        dmat = jax.lax.bitcast_convert_type(dmat32, jnp.bfloat16).reshape(PPAD, DP)
        loss = _head_loss(dmat, W1p, b1p, w2p, b2, tvec)[0, 0]
        total = loss if total is None else total + loss
    return total


# speedup vs baseline: 1.5027x; 1.5027x over previous
import functools

import jax
import jax.numpy as jnp
from jax import lax
from jax.experimental import pallas as pl
from jax.experimental.pallas import tpu as pltpu
from jax.experimental.pallas import tpu_sc as plsc

N = 10000
NPAD = 10240
E = 320000
P = 100000
DP = 128
DH = DP // 2
PPAD = 102400
NC, NS = 2, 16
NW = NC * NS
EPS = E // NS
RPT = NPAD // NS
PPW = PPAD // NW
KE = 80
KP = 80


def _linear(vf, Wp, bp):
    BM = 400

    def body(x_ref, w_ref, b_ref, o_ref):
        res = (
            jnp.dot(x_ref[...], w_ref[...], preferred_element_type=jnp.float32)
            + b_ref[...]
        )
        o_ref[0] = res[:, :DH]
        o_ref[1] = res[:, DH:]

    return pl.pallas_call(
        body,
        grid=(N // BM,),
        in_specs=[
            pl.BlockSpec((BM, DP), lambda i: (i, 0)),
            pl.BlockSpec((DP, DP), lambda i: (0, 0)),
            pl.BlockSpec((1, DP), lambda i: (0, 0)),
        ],
        out_specs=pl.BlockSpec((NC, BM, DH), lambda i: (0, i, 0)),
        out_shape=jax.ShapeDtypeStruct((NC, N, DH), jnp.float32),
    )(vf, Wp, bp)


def _edge_agg(h2, src, dst):
    mesh = plsc.VectorSubcoreMesh(core_axis_name="c", subcore_axis_name="s")
    nchunk = EPS // KE

    @functools.partial(
        pl.kernel,
        out_type=jax.ShapeDtypeStruct((NC, NPAD, DH), jnp.float32),
        mesh=mesh,
        compiler_params=pltpu.CompilerParams(use_tc_tiling_on_sc=False),
        scratch_types=[
            pltpu.VMEM((2, KE), jnp.int32),
            pltpu.VMEM((2, KE), jnp.int32),
            pltpu.VMEM((2, KE, DH), jnp.float32),
            pltpu.VMEM((RPT, DH), jnp.float32),
            pltpu.VMEM_SHARED((NPAD, DH), jnp.float32),
            pltpu.SemaphoreType.DMA,
            pltpu.SemaphoreType.DMA,
        ],
    )
    def k(h_hbm, src_hbm, dst_hbm, out_hbm, sidx, didx, rows, buf, acc,
          sem0, sem1):
        cid = lax.axis_index("c")
        sid = lax.axis_index("s")
        gsems = (sem0, sem1)

        @pl.loop(0, RPT, step=8)
        def _(r):
            for dr in range(8):
                for l in range(DH // 16):
                    buf[r + dr, pl.ds(l * 16, 16)] = jnp.zeros((16,), jnp.float32)

        pltpu.sync_copy(buf, acc.at[pl.ds(sid * RPT, RPT)])
        plsc.subcore_barrier()

        def fetch_and_fire(chunk, b):
            base = sid * EPS + chunk * KE
            pltpu.sync_copy(src_hbm.at[pl.ds(base, KE)], sidx.at[b])
            pltpu.sync_copy(dst_hbm.at[pl.ds(base, KE)], didx.at[b])
            pltpu.async_copy(h_hbm.at[cid].at[sidx.at[b]], rows.at[b], gsems[b])

        fetch_and_fire(0, 0)
        fetch_and_fire(1, 1)

        @pl.loop(0, nchunk, step=2)
        def _(it):
            for b in range(2):
                pltpu.make_async_copy(
                    h_hbm.at[cid].at[sidx.at[b]], rows.at[b], gsems[b]
                ).wait()
                pltpu.sync_copy(rows.at[b], acc.at[didx.at[b]], add=True)

                @pl.when(it + b + 2 < nchunk)
                def _():
                    fetch_and_fire(it + b + 2, b)

        plsc.subcore_barrier()
        pltpu.sync_copy(acc.at[pl.ds(sid * RPT, RPT)], buf)
        pltpu.sync_copy(buf, out_hbm.at[cid, pl.ds(sid * RPT, RPT)])

    return k(h2, src, dst)


def _finalize_gf(partials):
    BM = 512

    def body(p_ref, o_ref):
        x0 = p_ref[0]
        x1 = p_ref[1]
        li = lax.broadcasted_iota(jnp.int32, (BM, DH), 1)
        deg = jnp.sum(jnp.where(li == 100 - DH, x1, 0.0), axis=1, keepdims=True)
        r = 1.0 / jnp.maximum(deg, 1.0)
        g0 = jnp.maximum(x0 * r, 0.0)
        g1 = jnp.maximum(x1 * r, 0.0)

        def bf16_bits(x):
            u = jax.lax.bitcast_convert_type(x, jnp.int32)
            u = u + 0x7FFF + ((u >> 16) & 1)
            return jax.lax.shift_right_logical(u, 16)

        o_ref[...] = bf16_bits(g0) | jax.lax.shift_left(bf16_bits(g1), 16)

    return pl.pallas_call(
        body,
        grid=(NPAD // BM,),
        in_specs=[pl.BlockSpec((NC, BM, DH), lambda i: (0, i, 0))],
        out_specs=pl.BlockSpec((BM, DH), lambda i: (i, 0)),
        out_shape=jax.ShapeDtypeStruct((NPAD, DH), jnp.int32),
    )(partials)


def _pair_diff(gf32, pa, pb):
    mesh = plsc.VectorSubcoreMesh(core_axis_name="c", subcore_axis_name="s")
    nchunk = PPW // KP
    assert nchunk % 2 == 0 and nchunk >= 4
    W32 = DP // 2

    @functools.partial(
        pl.kernel,
        out_type=jax.ShapeDtypeStruct((PPAD, W32), jnp.int32),
        mesh=mesh,
        compiler_params=pltpu.CompilerParams(
            use_tc_tiling_on_sc=False, needs_layout_passes=False
        ),
        scratch_types=[
            pltpu.VMEM((2, KP), jnp.int32),
            pltpu.VMEM((2, KP), jnp.int32),
            pltpu.VMEM((2, KP, W32), jnp.int32),
            pltpu.VMEM((2, KP, W32), jnp.int32),
            pltpu.SemaphoreType.DMA,
            pltpu.SemaphoreType.DMA,
            pltpu.SemaphoreType.DMA,
            pltpu.SemaphoreType.DMA,
        ],
    )
    def k(gf_hbm, pa_hbm, pb_hbm, out_hbm, ia, ib, ra, rb,
          sa0, sa1, sb0, sb1):
        cid = lax.axis_index("c")
        sid = lax.axis_index("s")
        w = cid * NS + sid
        sA = (sa0, sa1)
        sB = (sb0, sb1)

        def fetch_and_fire(chunk, b):
            base = w * PPW + chunk * KP
            pltpu.sync_copy(pa_hbm.at[pl.ds(base, KP)], ia.at[b])
            pltpu.sync_copy(pb_hbm.at[pl.ds(base, KP)], ib.at[b])
            pltpu.async_copy(gf_hbm.at[ia.at[b]], ra.at[b], sA[b])
            pltpu.async_copy(gf_hbm.at[ib.at[b]], rb.at[b], sB[b])

        fetch_and_fire(0, 0)
        fetch_and_fire(1, 1)

        @pl.loop(0, nchunk, step=2)
        def _(it):
            for b in range(2):
                pltpu.make_async_copy(
                    gf_hbm.at[ia.at[b]], ra.at[b], sA[b]
                ).wait()
                pltpu.make_async_copy(
                    gf_hbm.at[ib.at[b]], rb.at[b], sB[b]
                ).wait()

                @pl.loop(0, KP, step=4)
                def _(r):
                    for dr in range(4):
                        for l in range(W32 // 16):
                            sl = pl.ds(l * 16, 16)
                            va = plsc.bitcast(ra[b, r + dr, sl], jnp.bfloat16)
                            vb = plsc.bitcast(rb[b, r + dr, sl], jnp.bfloat16)
                            ra[b, r + dr, sl] = plsc.bitcast(
                                jnp.abs(va - vb), jnp.int32
                            )

                base = w * PPW + (it + b) * KP
                pltpu.sync_copy(ra.at[b], out_hbm.at[pl.ds(base, KP)])

                @pl.when(it + b + 2 < nchunk)
                def _():
                    fetch_and_fire(it + b + 2, b)

    return k(gf32, pa, pb)


def _head_loss(dmat, W1lo, W1hi, b1p, w2p, b2, tvec):
    BM = 1024
    G = PPAD // BM
    DHID = 64
    SCALE = 1.0 / P

    def body(d_ref, w1lo_ref, w1hi_ref, b1_ref, w2_ref, b2_ref, t_ref, o_ref):
        i = pl.program_id(0)
        d32 = d_ref[...]
        dlo = jax.lax.bitcast_convert_type(
            jax.lax.shift_left(d32, 16), jnp.float32
        )
        dhi = jax.lax.bitcast_convert_type(
            d32 & jnp.int32(-65536), jnp.float32
        )
        hdn = jnp.maximum(
            jnp.dot(dlo, w1lo_ref[...], preferred_element_type=jnp.float32)
            + jnp.dot(dhi, w1hi_ref[...], preferred_element_type=jnp.float32)
            + b1_ref[...],
            0.0,
        )
        lg = jnp.sum(hdn * w2_ref[...], axis=1, keepdims=True) + b2_ref[...]
        t = t_ref[...]
        row = i * BM + lax.broadcasted_iota(jnp.int32, (BM, 1), 0)
        wgt = jnp.where(row < P, SCALE, 0.0)
        bce = jnp.maximum(lg, 0.0) - lg * t + jnp.log1p(jnp.exp(-jnp.abs(lg)))
        part = jnp.reshape(jnp.sum(bce * wgt), (1, 1))

        @pl.when(i == 0)
        def _():
            o_ref[...] = part

        @pl.when(i > 0)
        def _():
            o_ref[...] += part

    return pl.pallas_call(
        body,
        grid=(G,),
        in_specs=[
            pl.BlockSpec((BM, DP // 2), lambda i: (i, 0)),
            pl.BlockSpec((DP // 2, DHID), lambda i: (0, 0)),
            pl.BlockSpec((DP // 2, DHID), lambda i: (0, 0)),
            pl.BlockSpec((1, DHID), lambda i: (0, 0)),
            pl.BlockSpec((1, DHID), lambda i: (0, 0)),
            pl.BlockSpec((1, 1), lambda i: (0, 0)),
            pl.BlockSpec((BM, 1), lambda i: (i, 0)),
        ],
        out_specs=pl.BlockSpec((1, 1), lambda i: (0, 0)),
        out_shape=jax.ShapeDtypeStruct((1, 1), jnp.float32),
    )(dmat, W1lo, W1hi, b1p, w2p, b2, tvec)


def kernel(vertex_features, edge_index, pairs_cells, pairs_cols, pairs_rows,
           targets_cells, targets_cols, targets_rows,
           W_gcnn, b_gcnn, W_h1, b_h1, W_h2, b_h2):
    f32 = jnp.float32
    src = edge_index[0]
    dst = edge_index[1]

    Wp = jnp.pad(W_gcnn, ((0, 0), (0, DP - 100)))
    bp = jnp.concatenate(
        [b_gcnn, jnp.ones((1,), f32), jnp.zeros((DP - 101,), f32)]
    ).reshape(1, DP)
    W1lo = jnp.pad(W_h1[0:64], ((0, 0), (0, 14)))
    W1hi = jnp.pad(W_h1[64:100], ((0, 28), (0, 14)))
    b1p = jnp.pad(b_h1, (0, 14)).reshape(1, 64)
    w2p = jnp.pad(W_h2[:, 0], (0, 14)).reshape(1, 64)
    b2 = b_h2.reshape(1, 1)

    def padset(x):
        return jnp.pad(x, (0, PPAD - P))

    h2 = _linear(vertex_features, Wp, bp)
    partials = _edge_agg(h2, src, dst)
    gf32 = _finalize_gf(partials)

    total = None
    for pairs, targets in (
        (pairs_cells, targets_cells),
        (pairs_cols, targets_cols),
        (pairs_rows, targets_rows),
    ):
        pa = padset(pairs[:, 0])
        pb = padset(pairs[:, 1])
        tvec = padset(targets.astype(f32)).reshape(PPAD, 1)
        dmat32 = _pair_diff(gf32, pa, pb)
        loss = _head_loss(dmat32, W1lo, W1hi, b1p, w2p, b2, tvec)[0, 0]
        total = loss if total is None else total + loss
    return total


# speedup vs baseline: 1.7169x; 1.1425x over previous
import functools

import jax
import jax.numpy as jnp
from jax import lax
from jax.experimental import pallas as pl
from jax.experimental.pallas import tpu as pltpu
from jax.experimental.pallas import tpu_sc as plsc

N = 10000
NPAD = 10240
E = 320000
P = 100000
DP = 128
DH = DP // 2
W32 = DP // 2
PPAD = 102400
NC, NS = 2, 16
NW = NC * NS
EPS = E // NS
RPT = NPAD // NS
PPW = PPAD // NW
KE = 80
KP = 80
NCE = EPS // KE
NCP = PPW // KP


def _linear(vf, Wp, bp):
    BM = 400

    def body(x_ref, w_ref, b_ref, o_ref):
        res = (
            jnp.dot(x_ref[...], w_ref[...], preferred_element_type=jnp.float32)
            + b_ref[...]
        )
        o_ref[0] = res[:, :DH]
        o_ref[1] = res[:, DH:]

    return pl.pallas_call(
        body,
        grid=(N // BM,),
        in_specs=[
            pl.BlockSpec((BM, DP), lambda i: (i, 0)),
            pl.BlockSpec((DP, DP), lambda i: (0, 0)),
            pl.BlockSpec((1, DP), lambda i: (0, 0)),
        ],
        out_specs=pl.BlockSpec((NC, BM, DH), lambda i: (0, i, 0)),
        out_shape=jax.ShapeDtypeStruct((NC, N, DH), jnp.float32),
    )(vf, Wp, bp)


def _edge_agg(h2, ecomb):
    mesh = plsc.VectorSubcoreMesh(core_axis_name="c", subcore_axis_name="s")

    @functools.partial(
        pl.kernel,
        out_type=jax.ShapeDtypeStruct((NC, NPAD, DH), jnp.float32),
        mesh=mesh,
        compiler_params=pltpu.CompilerParams(use_tc_tiling_on_sc=False),
        scratch_types=[
            pltpu.VMEM((2 * NCE, KE), jnp.int32),
            pltpu.VMEM((2, KE, DH), jnp.float32),
            pltpu.VMEM_SHARED((NPAD, DH), jnp.float32),
            pltpu.SemaphoreType.DMA,
            pltpu.SemaphoreType.DMA,
            pltpu.SemaphoreType.DMA,
            pltpu.SemaphoreType.DMA,
            pltpu.SemaphoreType.DMA,
        ],
    )
    def k(h_hbm, e_hbm, out_hbm, idx, rows, acc,
          isem, g0, g1, s0, s1):
        cid = lax.axis_index("c")
        sid = lax.axis_index("s")
        gsem = (g0, g1)
        ssem = (s0, s1)

        pltpu.async_copy(
            e_hbm.at[pl.ds(sid * 2 * NCE, 2 * NCE)], idx, isem
        )

        @pl.loop(0, KE, step=8)
        def _(r):
            for dr in range(8):
                for l in range(DH // 16):
                    rows[0, r + dr, pl.ds(l * 16, 16)] = jnp.zeros(
                        (16,), jnp.float32
                    )

        for z in range(RPT // KE):
            pltpu.sync_copy(
                rows.at[0], acc.at[pl.ds(sid * RPT + z * KE, KE)]
            )
        pltpu.make_async_copy(
            e_hbm.at[pl.ds(sid * 2 * NCE, 2 * NCE)], idx, isem
        ).wait()
        plsc.subcore_barrier()

        def fire_gather(chunk, b):
            pltpu.async_copy(
                h_hbm.at[cid].at[idx.at[2 * chunk]], rows.at[b], gsem[b]
            )

        fire_gather(0, 0)
        fire_gather(1, 1)

        @pl.loop(0, NCE, step=2)
        def _(it):
            for b in range(2):
                chunk = it + b
                pltpu.make_async_copy(
                    h_hbm.at[cid].at[idx.at[0]], rows.at[b], gsem[b]
                ).wait()

                @pl.when(it >= 2)
                def _():
                    pltpu.make_async_copy(
                        rows.at[b], acc.at[idx.at[1]], ssem[b]
                    ).wait()

                pltpu.async_copy(
                    rows.at[b], acc.at[idx.at[2 * chunk + 1]], ssem[b],
                    add=True,
                )

                @pl.when(chunk + 2 < NCE)
                def _():
                    fire_gather(chunk + 2, b)

        for b in range(2):
            pltpu.make_async_copy(
                rows.at[b], acc.at[idx.at[1]], ssem[b]
            ).wait()
        plsc.subcore_barrier()
        for z in range(RPT // KE):
            pltpu.sync_copy(
                acc.at[pl.ds(sid * RPT + z * KE, KE)], rows.at[0]
            )
            pltpu.sync_copy(
                rows.at[0], out_hbm.at[cid, pl.ds(sid * RPT + z * KE, KE)]
            )

    return k(h2, ecomb)


def _finalize_gf(partials):
    BM = 512

    def body(p_ref, o_ref):
        x0 = p_ref[0]
        x1 = p_ref[1]
        li = lax.broadcasted_iota(jnp.int32, (BM, DH), 1)
        deg = jnp.sum(jnp.where(li == 100 - DH, x1, 0.0), axis=1, keepdims=True)
        r = 1.0 / jnp.maximum(deg, 1.0)
        g0 = jnp.maximum(x0 * r, 0.0)
        g1 = jnp.maximum(x1 * r, 0.0)

        def bf16_bits(x):
            u = jax.lax.bitcast_convert_type(x, jnp.int32)
            u = u + 0x7FFF + ((u >> 16) & 1)
            return jax.lax.shift_right_logical(u, 16)

        o_ref[...] = bf16_bits(g0) | jax.lax.shift_left(bf16_bits(g1), 16)

    return pl.pallas_call(
        body,
        grid=(NPAD // BM,),
        in_specs=[pl.BlockSpec((NC, BM, DH), lambda i: (0, i, 0))],
        out_specs=pl.BlockSpec((BM, DH), lambda i: (i, 0)),
        out_shape=jax.ShapeDtypeStruct((NPAD, DH), jnp.int32),
    )(partials)


def _pair_diff(gf32, pcomb):
    mesh = plsc.VectorSubcoreMesh(core_axis_name="c", subcore_axis_name="s")

    @functools.partial(
        pl.kernel,
        out_type=jax.ShapeDtypeStruct((PPAD, W32), jnp.int32),
        mesh=mesh,
        compiler_params=pltpu.CompilerParams(
            use_tc_tiling_on_sc=False, needs_layout_passes=False
        ),
        scratch_types=[
            pltpu.VMEM((2 * NCP, KP), jnp.int32),
            pltpu.VMEM((2, KP, W32), jnp.int32),
            pltpu.VMEM((2, KP, W32), jnp.int32),
            pltpu.VMEM((2, KP, W32), jnp.int32),
            pltpu.SemaphoreType.DMA,
            pltpu.SemaphoreType.DMA,
            pltpu.SemaphoreType.DMA,
            pltpu.SemaphoreType.DMA,
            pltpu.SemaphoreType.DMA,
            pltpu.SemaphoreType.DMA,
            pltpu.SemaphoreType.DMA,
        ],
    )
    def k(gf_hbm, p_hbm, out_hbm, idx, ra, rb, rc,
          isem, ga0, ga1, gb0, gb1, ss0, ss1):
        cid = lax.axis_index("c")
        sid = lax.axis_index("s")
        w = cid * NS + sid
        gA = (ga0, ga1)
        gB = (gb0, gb1)
        ssem = (ss0, ss1)

        pltpu.async_copy(
            p_hbm.at[pl.ds(w * 2 * NCP, 2 * NCP)], idx, isem
        )
        pltpu.make_async_copy(
            p_hbm.at[pl.ds(w * 2 * NCP, 2 * NCP)], idx, isem
        ).wait()

        def fire_gathers(chunk, b):
            pltpu.async_copy(gf_hbm.at[idx.at[2 * chunk]], ra.at[b], gA[b])
            pltpu.async_copy(gf_hbm.at[idx.at[2 * chunk + 1]], rb.at[b], gB[b])

        fire_gathers(0, 0)
        fire_gathers(1, 1)

        @pl.loop(0, NCP, step=2)
        def _(it):
            for b in range(2):
                chunk = it + b
                pltpu.make_async_copy(
                    gf_hbm.at[idx.at[0]], ra.at[b], gA[b]
                ).wait()
                pltpu.make_async_copy(
                    gf_hbm.at[idx.at[0]], rb.at[b], gB[b]
                ).wait()

                @pl.when(it >= 2)
                def _():
                    pltpu.make_async_copy(
                        rc.at[b], out_hbm.at[pl.ds(0, KP)], ssem[b]
                    ).wait()

                @pl.loop(0, KP, step=4)
                def _(r):
                    for dr in range(4):
                        for l in range(W32 // 16):
                            sl = pl.ds(l * 16, 16)
                            va = plsc.bitcast(ra[b, r + dr, sl], jnp.bfloat16)
                            vb = plsc.bitcast(rb[b, r + dr, sl], jnp.bfloat16)
                            rc[b, r + dr, sl] = plsc.bitcast(
                                jnp.abs(va - vb), jnp.int32
                            )

                @pl.when(chunk + 2 < NCP)
                def _():
                    fire_gathers(chunk + 2, b)

                base = w * PPW + chunk * KP
                pltpu.async_copy(
                    rc.at[b], out_hbm.at[pl.ds(base, KP)], ssem[b]
                )

        for b in range(2):
            pltpu.make_async_copy(
                rc.at[b], out_hbm.at[pl.ds(0, KP)], ssem[b]
            ).wait()

    return k(gf32, pcomb)


def _head_loss(dmat, W1lo, W1hi, b1p, w2p, b2, tvec):
    BM = 1024
    G = PPAD // BM
    DHID = 64
    SCALE = 1.0 / P

    def body(d_ref, w1lo_ref, w1hi_ref, b1_ref, w2_ref, b2_ref, t_ref, o_ref):
        i = pl.program_id(0)
        d32 = d_ref[...]
        dlo = jax.lax.bitcast_convert_type(
            jax.lax.shift_left(d32, 16), jnp.float32
        )
        dhi = jax.lax.bitcast_convert_type(
            d32 & jnp.int32(-65536), jnp.float32
        )
        hdn = jnp.maximum(
            jnp.dot(dlo, w1lo_ref[...], preferred_element_type=jnp.float32)
            + jnp.dot(dhi, w1hi_ref[...], preferred_element_type=jnp.float32)
            + b1_ref[...],
            0.0,
        )
        lg = jnp.sum(hdn * w2_ref[...], axis=1, keepdims=True) + b2_ref[...]
        t = t_ref[...]
        row = i * BM + lax.broadcasted_iota(jnp.int32, (BM, 1), 0)
        wgt = jnp.where(row < P, SCALE, 0.0)
        bce = jnp.maximum(lg, 0.0) - lg * t + jnp.log1p(jnp.exp(-jnp.abs(lg)))
        part = jnp.reshape(jnp.sum(bce * wgt), (1, 1))

        @pl.when(i == 0)
        def _():
            o_ref[...] = part

        @pl.when(i > 0)
        def _():
            o_ref[...] += part

    return pl.pallas_call(
        body,
        grid=(G,),
        in_specs=[
            pl.BlockSpec((BM, W32), lambda i: (i, 0)),
            pl.BlockSpec((W32, DHID), lambda i: (0, 0)),
            pl.BlockSpec((W32, DHID), lambda i: (0, 0)),
            pl.BlockSpec((1, DHID), lambda i: (0, 0)),
            pl.BlockSpec((1, DHID), lambda i: (0, 0)),
            pl.BlockSpec((1, 1), lambda i: (0, 0)),
            pl.BlockSpec((BM, 1), lambda i: (i, 0)),
        ],
        out_specs=pl.BlockSpec((1, 1), lambda i: (0, 0)),
        out_shape=jax.ShapeDtypeStruct((1, 1), jnp.float32),
    )(dmat, W1lo, W1hi, b1p, w2p, b2, tvec)


def kernel(vertex_features, edge_index, pairs_cells, pairs_cols, pairs_rows,
           targets_cells, targets_cols, targets_rows,
           W_gcnn, b_gcnn, W_h1, b_h1, W_h2, b_h2):
    f32 = jnp.float32
    src = edge_index[0]
    dst = edge_index[1]
    ecomb = jnp.stack(
        [src.reshape(NS, NCE, KE), dst.reshape(NS, NCE, KE)], axis=2
    ).reshape(NS * NCE * 2, KE)

    Wp = jnp.pad(W_gcnn, ((0, 0), (0, DP - 100)))
    bp = jnp.concatenate(
        [b_gcnn, jnp.ones((1,), f32), jnp.zeros((DP - 101,), f32)]
    ).reshape(1, DP)
    W1lo = jnp.pad(W_h1[0:64], ((0, 0), (0, 14)))
    W1hi = jnp.pad(W_h1[64:100], ((0, 28), (0, 14)))
    b1p = jnp.pad(b_h1, (0, 14)).reshape(1, 64)
    w2p = jnp.pad(W_h2[:, 0], (0, 14)).reshape(1, 64)
    b2 = b_h2.reshape(1, 1)

    def padset(x):
        return jnp.pad(x, (0, PPAD - P))

    h2 = _linear(vertex_features, Wp, bp)
    partials = _edge_agg(h2, ecomb)
    gf32 = _finalize_gf(partials)

    total = None
    for pairs, targets in (
        (pairs_cells, targets_cells),
        (pairs_cols, targets_cols),
        (pairs_rows, targets_rows),
    ):
        pa = padset(pairs[:, 0])
        pb = padset(pairs[:, 1])
        pcomb = jnp.stack(
            [pa.reshape(NW, NCP, KP), pb.reshape(NW, NCP, KP)], axis=2
        ).reshape(NW * NCP * 2, KP)
        tvec = padset(targets.astype(f32)).reshape(PPAD, 1)
        dmat32 = _pair_diff(gf32, pcomb)
        loss = _head_loss(dmat32, W1lo, W1hi, b1p, w2p, b2, tvec)[0, 0]
        total = loss if total is None else total + loss
    return total


# speedup vs baseline: 1.8007x; 1.0488x over previous
import functools

import jax
import jax.numpy as jnp
from jax import lax
from jax.experimental import pallas as pl
from jax.experimental.pallas import tpu as pltpu
from jax.experimental.pallas import tpu_sc as plsc

N = 10000
NPAD = 10240
E = 320000
P = 100000
DP = 128
DH = DP // 2
W32 = DP // 2
PPAD = 102400
PTOT = 3 * PPAD
NC, NS = 2, 16
NW = NC * NS
EPS = E // NS
RPT = NPAD // NS
PPW = PTOT // NW
KE = 80
KP = 80
NCE = EPS // KE
NCP = PPW // KP


def _linear(vf, Wp, bp):
    BM = 400

    def body(x_ref, w_ref, b_ref, o_ref):
        res = (
            jnp.dot(x_ref[...], w_ref[...], preferred_element_type=jnp.float32)
            + b_ref[...]
        )
        o_ref[0] = res[:, :DH]
        o_ref[1] = res[:, DH:]

    return pl.pallas_call(
        body,
        grid=(N // BM,),
        in_specs=[
            pl.BlockSpec((BM, DP), lambda i: (i, 0)),
            pl.BlockSpec((DP, DP), lambda i: (0, 0)),
            pl.BlockSpec((1, DP), lambda i: (0, 0)),
        ],
        out_specs=pl.BlockSpec((NC, BM, DH), lambda i: (0, i, 0)),
        out_shape=jax.ShapeDtypeStruct((NC, N, DH), jnp.float32),
    )(vf, Wp, bp)


def _edge_agg(h2, ecomb):
    mesh = plsc.VectorSubcoreMesh(core_axis_name="c", subcore_axis_name="s")

    @functools.partial(
        pl.kernel,
        out_type=jax.ShapeDtypeStruct((NC, NPAD, DH), jnp.float32),
        mesh=mesh,
        compiler_params=pltpu.CompilerParams(use_tc_tiling_on_sc=False),
        scratch_types=[
            pltpu.VMEM((2 * NCE, KE), jnp.int32),
            pltpu.VMEM((2, KE, DH), jnp.float32),
            pltpu.VMEM_SHARED((NPAD, DH), jnp.float32),
            pltpu.SemaphoreType.DMA,
            pltpu.SemaphoreType.DMA,
            pltpu.SemaphoreType.DMA,
            pltpu.SemaphoreType.DMA,
            pltpu.SemaphoreType.DMA,
        ],
    )
    def k(h_hbm, e_hbm, out_hbm, idx, rows, acc,
          isem, g0, g1, s0, s1):
        cid = lax.axis_index("c")
        sid = lax.axis_index("s")
        gsem = (g0, g1)
        ssem = (s0, s1)

        pltpu.async_copy(
            e_hbm.at[pl.ds(sid * 2 * NCE, 2 * NCE)], idx, isem
        )

        @pl.loop(0, KE, step=8)
        def _(r):
            for dr in range(8):
                for l in range(DH // 16):
                    rows[0, r + dr, pl.ds(l * 16, 16)] = jnp.zeros(
                        (16,), jnp.float32
                    )

        for z in range(RPT // KE):
            pltpu.sync_copy(
                rows.at[0], acc.at[pl.ds(sid * RPT + z * KE, KE)]
            )
        pltpu.make_async_copy(
            e_hbm.at[pl.ds(sid * 2 * NCE, 2 * NCE)], idx, isem
        ).wait()
        plsc.subcore_barrier()

        def fire_gather(chunk, b):
            pltpu.async_copy(
                h_hbm.at[cid].at[idx.at[2 * chunk]], rows.at[b], gsem[b]
            )

        fire_gather(0, 0)
        fire_gather(1, 1)

        @pl.loop(0, NCE, step=2)
        def _(it):
            for b in range(2):
                chunk = it + b
                pltpu.make_async_copy(
                    h_hbm.at[cid].at[idx.at[0]], rows.at[b], gsem[b]
                ).wait()

                @pl.when(it >= 2)
                def _():
                    pltpu.make_async_copy(
                        rows.at[b], acc.at[idx.at[1]], ssem[b]
                    ).wait()

                pltpu.async_copy(
                    rows.at[b], acc.at[idx.at[2 * chunk + 1]], ssem[b],
                    add=True,
                )

                @pl.when(chunk + 2 < NCE)
                def _():
                    fire_gather(chunk + 2, b)

        for b in range(2):
            pltpu.make_async_copy(
                rows.at[b], acc.at[idx.at[1]], ssem[b]
            ).wait()
        plsc.subcore_barrier()
        for z in range(RPT // KE):
            pltpu.sync_copy(
                acc.at[pl.ds(sid * RPT + z * KE, KE)], rows.at[0]
            )
            pltpu.sync_copy(
                rows.at[0], out_hbm.at[cid, pl.ds(sid * RPT + z * KE, KE)]
            )

    return k(h2, ecomb)


def _finalize_gf(partials):
    BM = 512

    def body(p_ref, o_ref):
        x0 = p_ref[0]
        x1 = p_ref[1]
        li = lax.broadcasted_iota(jnp.int32, (BM, DH), 1)
        deg = jnp.sum(jnp.where(li == 100 - DH, x1, 0.0), axis=1, keepdims=True)
        r = 1.0 / jnp.maximum(deg, 1.0)
        g0 = jnp.maximum(x0 * r, 0.0)
        g1 = jnp.maximum(x1 * r, 0.0)

        def bf16_bits(x):
            u = jax.lax.bitcast_convert_type(x, jnp.int32)
            u = u + 0x7FFF + ((u >> 16) & 1)
            return jax.lax.shift_right_logical(u, 16)

        o_ref[...] = bf16_bits(g0) | jax.lax.shift_left(bf16_bits(g1), 16)

    return pl.pallas_call(
        body,
        grid=(NPAD // BM,),
        in_specs=[pl.BlockSpec((NC, BM, DH), lambda i: (0, i, 0))],
        out_specs=pl.BlockSpec((BM, DH), lambda i: (i, 0)),
        out_shape=jax.ShapeDtypeStruct((NPAD, DH), jnp.int32),
    )(partials)


def _pair_diff(gf32, pcomb):
    mesh = plsc.VectorSubcoreMesh(core_axis_name="c", subcore_axis_name="s")

    @functools.partial(
        pl.kernel,
        out_type=jax.ShapeDtypeStruct((PTOT, W32), jnp.int32),
        mesh=mesh,
        compiler_params=pltpu.CompilerParams(
            use_tc_tiling_on_sc=False, needs_layout_passes=False
        ),
        scratch_types=[
            pltpu.VMEM((2 * NCP, KP), jnp.int32),
            pltpu.VMEM((2, KP, W32), jnp.int32),
            pltpu.VMEM((2, KP, W32), jnp.int32),
            pltpu.VMEM((2, KP, W32), jnp.int32),
            pltpu.SemaphoreType.DMA,
            pltpu.SemaphoreType.DMA,
            pltpu.SemaphoreType.DMA,
            pltpu.SemaphoreType.DMA,
            pltpu.SemaphoreType.DMA,
            pltpu.SemaphoreType.DMA,
            pltpu.SemaphoreType.DMA,
        ],
    )
    def k(gf_hbm, p_hbm, out_hbm, idx, ra, rb, rc,
          isem, ga0, ga1, gb0, gb1, ss0, ss1):
        cid = lax.axis_index("c")
        sid = lax.axis_index("s")
        w = cid * NS + sid
        gA = (ga0, ga1)
        gB = (gb0, gb1)
        ssem = (ss0, ss1)

        pltpu.async_copy(
            p_hbm.at[pl.ds(w * 2 * NCP, 2 * NCP)], idx, isem
        )
        pltpu.make_async_copy(
            p_hbm.at[pl.ds(w * 2 * NCP, 2 * NCP)], idx, isem
        ).wait()

        def fire_gathers(chunk, b):
            pltpu.async_copy(gf_hbm.at[idx.at[2 * chunk]], ra.at[b], gA[b])
            pltpu.async_copy(gf_hbm.at[idx.at[2 * chunk + 1]], rb.at[b], gB[b])

        fire_gathers(0, 0)
        fire_gathers(1, 1)

        @pl.loop(0, NCP, step=2)
        def _(it):
            for b in range(2):
                chunk = it + b
                pltpu.make_async_copy(
                    gf_hbm.at[idx.at[0]], ra.at[b], gA[b]
                ).wait()
                pltpu.make_async_copy(
                    gf_hbm.at[idx.at[0]], rb.at[b], gB[b]
                ).wait()

                @pl.when(it >= 2)
                def _():
                    pltpu.make_async_copy(
                        rc.at[b], out_hbm.at[pl.ds(0, KP)], ssem[b]
                    ).wait()

                @pl.loop(0, KP, step=4)
                def _(r):
                    for dr in range(4):
                        for l in range(W32 // 16):
                            sl = pl.ds(l * 16, 16)
                            va = plsc.bitcast(ra[b, r + dr, sl], jnp.bfloat16)
                            vb = plsc.bitcast(rb[b, r + dr, sl], jnp.bfloat16)
                            rc[b, r + dr, sl] = plsc.bitcast(
                                jnp.abs(va - vb), jnp.int32
                            )

                @pl.when(chunk + 2 < NCP)
                def _():
                    fire_gathers(chunk + 2, b)

                base = w * PPW + chunk * KP
                pltpu.async_copy(
                    rc.at[b], out_hbm.at[pl.ds(base, KP)], ssem[b]
                )

        for b in range(2):
            pltpu.make_async_copy(
                rc.at[b], out_hbm.at[pl.ds(0, KP)], ssem[b]
            ).wait()

    return k(gf32, pcomb)


def _head_loss(dmat, W1lo, W1hi, b1p, w2p, b2, tvec):
    BM = 1024
    G = PTOT // BM
    DHID = 64
    SCALE = 1.0 / P

    def body(d_ref, w1lo_ref, w1hi_ref, b1_ref, w2_ref, b2_ref, t_ref, o_ref):
        i = pl.program_id(0)
        d32 = d_ref[...]
        dlo = jax.lax.bitcast_convert_type(
            jax.lax.shift_left(d32, 16), jnp.float32
        )
        dhi = jax.lax.bitcast_convert_type(
            d32 & jnp.int32(-65536), jnp.float32
        )
        hdn = jnp.maximum(
            jnp.dot(dlo, w1lo_ref[...], preferred_element_type=jnp.float32)
            + jnp.dot(dhi, w1hi_ref[...], preferred_element_type=jnp.float32)
            + b1_ref[...],
            0.0,
        )
        lg = jnp.sum(hdn * w2_ref[...], axis=1, keepdims=True) + b2_ref[...]
        t = t_ref[...]
        row = i * BM + lax.broadcasted_iota(jnp.int32, (BM, 1), 0)
        sidx = (row >= PPAD).astype(jnp.int32) + (row >= 2 * PPAD).astype(
            jnp.int32
        )
        wgt = jnp.where(row - sidx * PPAD < P, SCALE, 0.0)
        bce = jnp.maximum(lg, 0.0) - lg * t + jnp.log1p(jnp.exp(-jnp.abs(lg)))
        part = jnp.reshape(jnp.sum(bce * wgt), (1, 1))

        @pl.when(i == 0)
        def _():
            o_ref[...] = part

        @pl.when(i > 0)
        def _():
            o_ref[...] += part

    return pl.pallas_call(
        body,
        grid=(G,),
        in_specs=[
            pl.BlockSpec((BM, W32), lambda i: (i, 0)),
            pl.BlockSpec((W32, DHID), lambda i: (0, 0)),
            pl.BlockSpec((W32, DHID), lambda i: (0, 0)),
            pl.BlockSpec((1, DHID), lambda i: (0, 0)),
            pl.BlockSpec((1, DHID), lambda i: (0, 0)),
            pl.BlockSpec((1, 1), lambda i: (0, 0)),
            pl.BlockSpec((BM, 1), lambda i: (i, 0)),
        ],
        out_specs=pl.BlockSpec((1, 1), lambda i: (0, 0)),
        out_shape=jax.ShapeDtypeStruct((1, 1), jnp.float32),
    )(dmat, W1lo, W1hi, b1p, w2p, b2, tvec)


def kernel(vertex_features, edge_index, pairs_cells, pairs_cols, pairs_rows,
           targets_cells, targets_cols, targets_rows,
           W_gcnn, b_gcnn, W_h1, b_h1, W_h2, b_h2):
    f32 = jnp.float32
    src = edge_index[0]
    dst = edge_index[1]
    ecomb = jnp.stack(
        [src.reshape(NS, NCE, KE), dst.reshape(NS, NCE, KE)], axis=2
    ).reshape(NS * NCE * 2, KE)

    Wp = jnp.pad(W_gcnn, ((0, 0), (0, DP - 100)))
    bp = jnp.concatenate(
        [b_gcnn, jnp.ones((1,), f32), jnp.zeros((DP - 101,), f32)]
    ).reshape(1, DP)
    W1lo = jnp.pad(W_h1[0:64], ((0, 0), (0, 14)))
    W1hi = jnp.pad(W_h1[64:100], ((0, 28), (0, 14)))
    b1p = jnp.pad(b_h1, (0, 14)).reshape(1, 64)
    w2p = jnp.pad(W_h2[:, 0], (0, 14)).reshape(1, 64)
    b2 = b_h2.reshape(1, 1)

    def padset(x):
        return jnp.pad(x, (0, PPAD - P))

    h2 = _linear(vertex_features, Wp, bp)
    partials = _edge_agg(h2, ecomb)
    gf32 = _finalize_gf(partials)

    pa = jnp.concatenate(
        [padset(pairs_cells[:, 0]), padset(pairs_cols[:, 0]),
         padset(pairs_rows[:, 0])]
    )
    pb = jnp.concatenate(
        [padset(pairs_cells[:, 1]), padset(pairs_cols[:, 1]),
         padset(pairs_rows[:, 1])]
    )
    pcomb = jnp.stack(
        [pa.reshape(NW, NCP, KP), pb.reshape(NW, NCP, KP)], axis=2
    ).reshape(NW * NCP * 2, KP)
    tvec = jnp.concatenate(
        [
            padset(targets_cells.astype(f32)),
            padset(targets_cols.astype(f32)),
            padset(targets_rows.astype(f32)),
        ]
    ).reshape(PTOT, 1)
    dmat32 = _pair_diff(gf32, pcomb)
    return _head_loss(dmat32, W1lo, W1hi, b1p, w2p, b2, tvec)[0, 0]


# speedup vs baseline: 1.8477x; 1.0261x over previous
import functools

import jax
import jax.numpy as jnp
from jax import lax
from jax.experimental import pallas as pl
from jax.experimental.pallas import tpu as pltpu
from jax.experimental.pallas import tpu_sc as plsc

N = 10000
NPAD = 10240
E = 320000
P = 100000
DP = 128
DH = DP // 2
W32 = DP // 2
PPAD = 102400
PTOT = 3 * PPAD
NC, NS = 2, 16
NW = NC * NS
EPS = E // NS
RPT = NPAD // NS
PPW = PTOT // NW
KE = 80
KP = 96
NCE = EPS // KE
NCP = PPW // KP


def _linear(vf, Wp, bp):
    BM = 400

    def body(x_ref, w_ref, b_ref, o_ref):
        res = (
            jnp.dot(x_ref[...], w_ref[...], preferred_element_type=jnp.float32)
            + b_ref[...]
        )
        o_ref[0] = res[:, :DH]
        o_ref[1] = res[:, DH:]

    return pl.pallas_call(
        body,
        grid=(N // BM,),
        in_specs=[
            pl.BlockSpec((BM, DP), lambda i: (i, 0)),
            pl.BlockSpec((DP, DP), lambda i: (0, 0)),
            pl.BlockSpec((1, DP), lambda i: (0, 0)),
        ],
        out_specs=pl.BlockSpec((NC, BM, DH), lambda i: (0, i, 0)),
        out_shape=jax.ShapeDtypeStruct((NC, N, DH), jnp.float32),
    )(vf, Wp, bp)


def _edge_agg(h2, ecomb):
    mesh = plsc.VectorSubcoreMesh(core_axis_name="c", subcore_axis_name="s")

    @functools.partial(
        pl.kernel,
        out_type=jax.ShapeDtypeStruct((NC, NPAD, DH), jnp.float32),
        mesh=mesh,
        compiler_params=pltpu.CompilerParams(use_tc_tiling_on_sc=False),
        scratch_types=[
            pltpu.VMEM((2 * NCE, KE), jnp.int32),
            pltpu.VMEM((2, KE, DH), jnp.float32),
            pltpu.VMEM_SHARED((NPAD, DH), jnp.float32),
            pltpu.SemaphoreType.DMA,
            pltpu.SemaphoreType.DMA,
            pltpu.SemaphoreType.DMA,
            pltpu.SemaphoreType.DMA,
            pltpu.SemaphoreType.DMA,
        ],
    )
    def k(h_hbm, e_hbm, out_hbm, idx, rows, acc,
          isem, g0, g1, s0, s1):
        cid = lax.axis_index("c")
        sid = lax.axis_index("s")
        gsem = (g0, g1)
        ssem = (s0, s1)

        pltpu.async_copy(
            e_hbm.at[pl.ds(sid * 2 * NCE, 2 * NCE)], idx, isem
        )

        @pl.loop(0, KE, step=8)
        def _(r):
            for dr in range(8):
                for l in range(DH // 16):
                    rows[0, r + dr, pl.ds(l * 16, 16)] = jnp.zeros(
                        (16,), jnp.float32
                    )

        for z in range(RPT // KE):
            pltpu.sync_copy(
                rows.at[0], acc.at[pl.ds(sid * RPT + z * KE, KE)]
            )
        pltpu.make_async_copy(
            e_hbm.at[pl.ds(sid * 2 * NCE, 2 * NCE)], idx, isem
        ).wait()
        plsc.subcore_barrier()

        def fire_gather(chunk, b):
            pltpu.async_copy(
                h_hbm.at[cid].at[idx.at[2 * chunk]], rows.at[b], gsem[b]
            )

        fire_gather(0, 0)
        fire_gather(1, 1)

        @pl.loop(0, NCE, step=2)
        def _(it):
            for b in range(2):
                chunk = it + b
                pltpu.make_async_copy(
                    h_hbm.at[cid].at[idx.at[0]], rows.at[b], gsem[b]
                ).wait()

                @pl.when(it >= 2)
                def _():
                    pltpu.make_async_copy(
                        rows.at[b], acc.at[idx.at[1]], ssem[b]
                    ).wait()

                pltpu.async_copy(
                    rows.at[b], acc.at[idx.at[2 * chunk + 1]], ssem[b],
                    add=True,
                )

                @pl.when(chunk + 2 < NCE)
                def _():
                    fire_gather(chunk + 2, b)

        for b in range(2):
            pltpu.make_async_copy(
                rows.at[b], acc.at[idx.at[1]], ssem[b]
            ).wait()
        plsc.subcore_barrier()
        for z in range(RPT // KE):
            pltpu.sync_copy(
                acc.at[pl.ds(sid * RPT + z * KE, KE)], rows.at[0]
            )
            pltpu.sync_copy(
                rows.at[0], out_hbm.at[cid, pl.ds(sid * RPT + z * KE, KE)]
            )

    return k(h2, ecomb)


def _finalize_gf(partials):
    BM = 512

    def body(p_ref, o_ref):
        x0 = p_ref[0]
        x1 = p_ref[1]
        li = lax.broadcasted_iota(jnp.int32, (BM, DH), 1)
        deg = jnp.sum(jnp.where(li == 100 - DH, x1, 0.0), axis=1, keepdims=True)
        r = 1.0 / jnp.maximum(deg, 1.0)
        g0 = jnp.maximum(x0 * r, 0.0)
        g1 = jnp.maximum(x1 * r, 0.0)

        def bf16_bits(x):
            u = jax.lax.bitcast_convert_type(x, jnp.int32)
            u = u + 0x7FFF + ((u >> 16) & 1)
            return jax.lax.shift_right_logical(u, 16)

        o_ref[...] = bf16_bits(g0) | jax.lax.shift_left(bf16_bits(g1), 16)

    return pl.pallas_call(
        body,
        grid=(NPAD // BM,),
        in_specs=[pl.BlockSpec((NC, BM, DH), lambda i: (0, i, 0))],
        out_specs=pl.BlockSpec((BM, DH), lambda i: (i, 0)),
        out_shape=jax.ShapeDtypeStruct((NPAD, DH), jnp.int32),
    )(partials)


def _pair_diff(gf32, pcomb):
    mesh = plsc.VectorSubcoreMesh(core_axis_name="c", subcore_axis_name="s")

    @functools.partial(
        pl.kernel,
        out_type=jax.ShapeDtypeStruct((PTOT, W32), jnp.int32),
        mesh=mesh,
        compiler_params=pltpu.CompilerParams(
            use_tc_tiling_on_sc=False, needs_layout_passes=False
        ),
        scratch_types=[
            pltpu.VMEM((2 * NCP, KP), jnp.int32),
            pltpu.VMEM((4, KP, W32), jnp.int32),
            pltpu.VMEM((4, KP, W32), jnp.int32),
            pltpu.VMEM((4, KP, W32), jnp.int32),
            pltpu.SemaphoreType.DMA,
            pltpu.SemaphoreType.DMA,
            pltpu.SemaphoreType.DMA,
            pltpu.SemaphoreType.DMA,
            pltpu.SemaphoreType.DMA,
            pltpu.SemaphoreType.DMA,
            pltpu.SemaphoreType.DMA,
            pltpu.SemaphoreType.DMA,
            pltpu.SemaphoreType.DMA,
            pltpu.SemaphoreType.DMA,
            pltpu.SemaphoreType.DMA,
            pltpu.SemaphoreType.DMA,
            pltpu.SemaphoreType.DMA,
        ],
    )
    def k(gf_hbm, p_hbm, out_hbm, idx, ra, rb, rc,
          isem, ga0, ga1, ga2, ga3, gb0, gb1, gb2, gb3,
          ss0, ss1, ss2, ss3):
        cid = lax.axis_index("c")
        sid = lax.axis_index("s")
        w = cid * NS + sid
        gA = (ga0, ga1, ga2, ga3)
        gB = (gb0, gb1, gb2, gb3)
        ssem = (ss0, ss1, ss2, ss3)

        pltpu.async_copy(
            p_hbm.at[pl.ds(w * 2 * NCP, 2 * NCP)], idx, isem
        )
        pltpu.make_async_copy(
            p_hbm.at[pl.ds(w * 2 * NCP, 2 * NCP)], idx, isem
        ).wait()

        def fire_gathers(chunk, b):
            pltpu.async_copy(gf_hbm.at[idx.at[2 * chunk]], ra.at[b], gA[b])
            pltpu.async_copy(gf_hbm.at[idx.at[2 * chunk + 1]], rb.at[b], gB[b])

        for b in range(4):
            fire_gathers(b, b)

        @pl.loop(0, NCP, step=4)
        def _(it):
            for b in range(4):
                chunk = it + b
                pltpu.make_async_copy(
                    gf_hbm.at[idx.at[0]], ra.at[b], gA[b]
                ).wait()
                pltpu.make_async_copy(
                    gf_hbm.at[idx.at[0]], rb.at[b], gB[b]
                ).wait()

                @pl.when(it >= 4)
                def _():
                    pltpu.make_async_copy(
                        rc.at[b], out_hbm.at[pl.ds(0, KP)], ssem[b]
                    ).wait()

                @pl.loop(0, KP, step=4)
                def _(r):
                    for dr in range(4):
                        for l in range(W32 // 16):
                            sl = pl.ds(l * 16, 16)
                            va = plsc.bitcast(ra[b, r + dr, sl], jnp.bfloat16)
                            vb = plsc.bitcast(rb[b, r + dr, sl], jnp.bfloat16)
                            rc[b, r + dr, sl] = plsc.bitcast(
                                jnp.abs(va - vb), jnp.int32
                            )

                @pl.when(chunk + 4 < NCP)
                def _():
                    fire_gathers(chunk + 4, b)

                base = w * PPW + chunk * KP
                pltpu.async_copy(
                    rc.at[b], out_hbm.at[pl.ds(base, KP)], ssem[b]
                )

        for b in range(4):
            pltpu.make_async_copy(
                rc.at[b], out_hbm.at[pl.ds(0, KP)], ssem[b]
            ).wait()

    return k(gf32, pcomb)


def _head_loss(dmat, W1lo, W1hi, b1p, w2p, b2, tvec):
    BM = 1024
    G = PTOT // BM
    DHID = 64
    SCALE = 1.0 / P

    def body(d_ref, w1lo_ref, w1hi_ref, b1_ref, w2_ref, b2_ref, t_ref, o_ref):
        i = pl.program_id(0)
        d32 = d_ref[...]
        dlo = jax.lax.bitcast_convert_type(
            jax.lax.shift_left(d32, 16), jnp.float32
        )
        dhi = jax.lax.bitcast_convert_type(
            d32 & jnp.int32(-65536), jnp.float32
        )
        hdn = jnp.maximum(
            jnp.dot(dlo, w1lo_ref[...], preferred_element_type=jnp.float32)
            + jnp.dot(dhi, w1hi_ref[...], preferred_element_type=jnp.float32)
            + b1_ref[...],
            0.0,
        )
        lg = jnp.sum(hdn * w2_ref[...], axis=1, keepdims=True) + b2_ref[...]
        t = t_ref[...]
        row = i * BM + lax.broadcasted_iota(jnp.int32, (BM, 1), 0)
        sidx = (row >= PPAD).astype(jnp.int32) + (row >= 2 * PPAD).astype(
            jnp.int32
        )
        wgt = jnp.where(row - sidx * PPAD < P, SCALE, 0.0)
        bce = jnp.maximum(lg, 0.0) - lg * t + jnp.log1p(jnp.exp(-jnp.abs(lg)))
        part = jnp.reshape(jnp.sum(bce * wgt), (1, 1))

        @pl.when(i == 0)
        def _():
            o_ref[...] = part

        @pl.when(i > 0)
        def _():
            o_ref[...] += part

    return pl.pallas_call(
        body,
        grid=(G,),
        in_specs=[
            pl.BlockSpec((BM, W32), lambda i: (i, 0)),
            pl.BlockSpec((W32, DHID), lambda i: (0, 0)),
            pl.BlockSpec((W32, DHID), lambda i: (0, 0)),
            pl.BlockSpec((1, DHID), lambda i: (0, 0)),
            pl.BlockSpec((1, DHID), lambda i: (0, 0)),
            pl.BlockSpec((1, 1), lambda i: (0, 0)),
            pl.BlockSpec((BM, 1), lambda i: (i, 0)),
        ],
        out_specs=pl.BlockSpec((1, 1), lambda i: (0, 0)),
        out_shape=jax.ShapeDtypeStruct((1, 1), jnp.float32),
    )(dmat, W1lo, W1hi, b1p, w2p, b2, tvec)


def kernel(vertex_features, edge_index, pairs_cells, pairs_cols, pairs_rows,
           targets_cells, targets_cols, targets_rows,
           W_gcnn, b_gcnn, W_h1, b_h1, W_h2, b_h2):
    f32 = jnp.float32
    src = edge_index[0]
    dst = edge_index[1]
    ecomb = jnp.stack(
        [src.reshape(NS, NCE, KE), dst.reshape(NS, NCE, KE)], axis=2
    ).reshape(NS * NCE * 2, KE)

    Wp = jnp.pad(W_gcnn, ((0, 0), (0, DP - 100)))
    bp = jnp.concatenate(
        [b_gcnn, jnp.ones((1,), f32), jnp.zeros((DP - 101,), f32)]
    ).reshape(1, DP)
    W1lo = jnp.pad(W_h1[0:64], ((0, 0), (0, 14)))
    W1hi = jnp.pad(W_h1[64:100], ((0, 28), (0, 14)))
    b1p = jnp.pad(b_h1, (0, 14)).reshape(1, 64)
    w2p = jnp.pad(W_h2[:, 0], (0, 14)).reshape(1, 64)
    b2 = b_h2.reshape(1, 1)

    def padset(x):
        return jnp.pad(x, (0, PPAD - P))

    h2 = _linear(vertex_features, Wp, bp)
    partials = _edge_agg(h2, ecomb)
    gf32 = _finalize_gf(partials)

    pa = jnp.concatenate(
        [padset(pairs_cells[:, 0]), padset(pairs_cols[:, 0]),
         padset(pairs_rows[:, 0])]
    )
    pb = jnp.concatenate(
        [padset(pairs_cells[:, 1]), padset(pairs_cols[:, 1]),
         padset(pairs_rows[:, 1])]
    )
    pcomb = jnp.stack(
        [pa.reshape(NW, NCP, KP), pb.reshape(NW, NCP, KP)], axis=2
    ).reshape(NW * NCP * 2, KP)
    tvec = jnp.concatenate(
        [
            padset(targets_cells.astype(f32)),
            padset(targets_cols.astype(f32)),
            padset(targets_rows.astype(f32)),
        ]
    ).reshape(PTOT, 1)
    dmat32 = _pair_diff(gf32, pcomb)
    return _head_loss(dmat32, W1lo, W1hi, b1p, w2p, b2, tvec)[0, 0]


# speedup vs baseline: 1.8484x; 1.0004x over previous
import functools

import jax
import jax.numpy as jnp
from jax import lax
from jax.experimental import pallas as pl
from jax.experimental.pallas import tpu as pltpu
from jax.experimental.pallas import tpu_sc as plsc

N = 10000
NPAD = 10240
E = 320000
P = 100000
DP = 128
DH = DP // 2
W32 = DP // 2
PPAD = 102400
PTOT = 3 * PPAD
NC, NS = 2, 16
NW = NC * NS
EPS = E // NS
RPT = NPAD // NS
PPW = PTOT // NW
KE = 80
KP = 120
NCE = EPS // KE
NCP = PPW // KP


def _linear(vf, Wp, bp):
    BM = 400

    def body(x_ref, w_ref, b_ref, o_ref):
        res = (
            jnp.dot(x_ref[...], w_ref[...], preferred_element_type=jnp.float32)
            + b_ref[...]
        )
        o_ref[0] = res[:, :DH]
        o_ref[1] = res[:, DH:]

    return pl.pallas_call(
        body,
        grid=(N // BM,),
        in_specs=[
            pl.BlockSpec((BM, DP), lambda i: (i, 0)),
            pl.BlockSpec((DP, DP), lambda i: (0, 0)),
            pl.BlockSpec((1, DP), lambda i: (0, 0)),
        ],
        out_specs=pl.BlockSpec((NC, BM, DH), lambda i: (0, i, 0)),
        out_shape=jax.ShapeDtypeStruct((NC, N, DH), jnp.float32),
    )(vf, Wp, bp)


def _edge_agg(h2, ecomb):
    mesh = plsc.VectorSubcoreMesh(core_axis_name="c", subcore_axis_name="s")

    @functools.partial(
        pl.kernel,
        out_type=jax.ShapeDtypeStruct((NC, NPAD, DH), jnp.float32),
        mesh=mesh,
        compiler_params=pltpu.CompilerParams(use_tc_tiling_on_sc=False),
        scratch_types=[
            pltpu.VMEM((2 * NCE, KE), jnp.int32),
            pltpu.VMEM((2, KE, DH), jnp.float32),
            pltpu.VMEM_SHARED((NPAD, DH), jnp.float32),
            pltpu.SemaphoreType.DMA,
            pltpu.SemaphoreType.DMA,
            pltpu.SemaphoreType.DMA,
            pltpu.SemaphoreType.DMA,
            pltpu.SemaphoreType.DMA,
        ],
    )
    def k(h_hbm, e_hbm, out_hbm, idx, rows, acc,
          isem, g0, g1, s0, s1):
        cid = lax.axis_index("c")
        sid = lax.axis_index("s")
        gsem = (g0, g1)
        ssem = (s0, s1)

        pltpu.async_copy(
            e_hbm.at[pl.ds(sid * 2 * NCE, 2 * NCE)], idx, isem
        )

        @pl.loop(0, KE, step=8)
        def _(r):
            for dr in range(8):
                for l in range(DH // 16):
                    rows[0, r + dr, pl.ds(l * 16, 16)] = jnp.zeros(
                        (16,), jnp.float32
                    )

        for z in range(RPT // KE):
            pltpu.sync_copy(
                rows.at[0], acc.at[pl.ds(sid * RPT + z * KE, KE)]
            )
        pltpu.make_async_copy(
            e_hbm.at[pl.ds(sid * 2 * NCE, 2 * NCE)], idx, isem
        ).wait()
        plsc.subcore_barrier()

        def fire_gather(chunk, b):
            pltpu.async_copy(
                h_hbm.at[cid].at[idx.at[2 * chunk]], rows.at[b], gsem[b]
            )

        fire_gather(0, 0)
        fire_gather(1, 1)

        @pl.loop(0, NCE, step=2)
        def _(it):
            for b in range(2):
                chunk = it + b
                pltpu.make_async_copy(
                    h_hbm.at[cid].at[idx.at[0]], rows.at[b], gsem[b]
                ).wait()

                @pl.when(it >= 2)
                def _():
                    pltpu.make_async_copy(
                        rows.at[b], acc.at[idx.at[1]], ssem[b]
                    ).wait()

                pltpu.async_copy(
                    rows.at[b], acc.at[idx.at[2 * chunk + 1]], ssem[b],
                    add=True,
                )

                @pl.when(chunk + 2 < NCE)
                def _():
                    fire_gather(chunk + 2, b)

        for b in range(2):
            pltpu.make_async_copy(
                rows.at[b], acc.at[idx.at[1]], ssem[b]
            ).wait()
        plsc.subcore_barrier()
        for z in range(RPT // KE):
            pltpu.sync_copy(
                acc.at[pl.ds(sid * RPT + z * KE, KE)], rows.at[0]
            )
            pltpu.sync_copy(
                rows.at[0], out_hbm.at[cid, pl.ds(sid * RPT + z * KE, KE)]
            )

    return k(h2, ecomb)


def _finalize_gf(partials):
    BM = 512

    def body(p_ref, o_ref):
        x0 = p_ref[0]
        x1 = p_ref[1]
        li = lax.broadcasted_iota(jnp.int32, (BM, DH), 1)
        deg = jnp.sum(jnp.where(li == 100 - DH, x1, 0.0), axis=1, keepdims=True)
        r = 1.0 / jnp.maximum(deg, 1.0)
        g0 = jnp.maximum(x0 * r, 0.0)
        g1 = jnp.maximum(x1 * r, 0.0)

        def bf16_bits(x):
            u = jax.lax.bitcast_convert_type(x, jnp.int32)
            u = u + 0x7FFF + ((u >> 16) & 1)
            return jax.lax.shift_right_logical(u, 16)

        o_ref[...] = bf16_bits(g0) | jax.lax.shift_left(bf16_bits(g1), 16)

    return pl.pallas_call(
        body,
        grid=(NPAD // BM,),
        in_specs=[pl.BlockSpec((NC, BM, DH), lambda i: (0, i, 0))],
        out_specs=pl.BlockSpec((BM, DH), lambda i: (i, 0)),
        out_shape=jax.ShapeDtypeStruct((NPAD, DH), jnp.int32),
    )(partials)


def _pair_diff(gf32, pcomb):
    mesh = plsc.VectorSubcoreMesh(core_axis_name="c", subcore_axis_name="s")

    @functools.partial(
        pl.kernel,
        out_type=jax.ShapeDtypeStruct((PTOT, W32), jnp.int32),
        mesh=mesh,
        compiler_params=pltpu.CompilerParams(
            use_tc_tiling_on_sc=False, needs_layout_passes=False
        ),
        scratch_types=[
            pltpu.VMEM((2 * NCP, KP), jnp.int32),
            pltpu.VMEM((4, KP, W32), jnp.int32),
            pltpu.VMEM((4, KP, W32), jnp.int32),
            pltpu.VMEM((4, KP, W32), jnp.int32),
            pltpu.SemaphoreType.DMA,
            pltpu.SemaphoreType.DMA,
            pltpu.SemaphoreType.DMA,
            pltpu.SemaphoreType.DMA,
            pltpu.SemaphoreType.DMA,
            pltpu.SemaphoreType.DMA,
            pltpu.SemaphoreType.DMA,
            pltpu.SemaphoreType.DMA,
            pltpu.SemaphoreType.DMA,
            pltpu.SemaphoreType.DMA,
            pltpu.SemaphoreType.DMA,
            pltpu.SemaphoreType.DMA,
            pltpu.SemaphoreType.DMA,
        ],
    )
    def k(gf_hbm, p_hbm, out_hbm, idx, ra, rb, rc,
          isem, ga0, ga1, ga2, ga3, gb0, gb1, gb2, gb3,
          ss0, ss1, ss2, ss3):
        cid = lax.axis_index("c")
        sid = lax.axis_index("s")
        w = cid * NS + sid
        gA = (ga0, ga1, ga2, ga3)
        gB = (gb0, gb1, gb2, gb3)
        ssem = (ss0, ss1, ss2, ss3)

        pltpu.async_copy(
            p_hbm.at[pl.ds(w * 2 * NCP, 2 * NCP)], idx, isem
        )
        pltpu.make_async_copy(
            p_hbm.at[pl.ds(w * 2 * NCP, 2 * NCP)], idx, isem
        ).wait()

        def fire_gathers(chunk, b):
            pltpu.async_copy(gf_hbm.at[idx.at[2 * chunk]], ra.at[b], gA[b])
            pltpu.async_copy(gf_hbm.at[idx.at[2 * chunk + 1]], rb.at[b], gB[b])

        for b in range(4):
            fire_gathers(b, b)

        @pl.loop(0, NCP, step=4)
        def _(it):
            for b in range(4):
                chunk = it + b
                pltpu.make_async_copy(
                    gf_hbm.at[idx.at[0]], ra.at[b], gA[b]
                ).wait()
                pltpu.make_async_copy(
                    gf_hbm.at[idx.at[0]], rb.at[b], gB[b]
                ).wait()

                @pl.when(it >= 4)
                def _():
                    pltpu.make_async_copy(
                        rc.at[b], out_hbm.at[pl.ds(0, KP)], ssem[b]
                    ).wait()

                @pl.loop(0, KP, step=4)
                def _(r):
                    for dr in range(4):
                        for l in range(W32 // 16):
                            sl = pl.ds(l * 16, 16)
                            va = plsc.bitcast(ra[b, r + dr, sl], jnp.bfloat16)
                            vb = plsc.bitcast(rb[b, r + dr, sl], jnp.bfloat16)
                            rc[b, r + dr, sl] = plsc.bitcast(
                                jnp.abs(va - vb), jnp.int32
                            )

                @pl.when(chunk + 4 < NCP)
                def _():
                    fire_gathers(chunk + 4, b)

                base = w * PPW + chunk * KP
                pltpu.async_copy(
                    rc.at[b], out_hbm.at[pl.ds(base, KP)], ssem[b]
                )

        for b in range(4):
            pltpu.make_async_copy(
                rc.at[b], out_hbm.at[pl.ds(0, KP)], ssem[b]
            ).wait()

    return k(gf32, pcomb)


def _head_loss(dmat, W1lo, W1hi, b1p, w2p, b2, tvec):
    BM = 1024
    G = PTOT // BM
    DHID = 64
    SCALE = 1.0 / P

    def body(d_ref, w1lo_ref, w1hi_ref, b1_ref, w2_ref, b2_ref, t_ref, o_ref):
        i = pl.program_id(0)
        d32 = d_ref[...]
        dlo = jax.lax.bitcast_convert_type(
            jax.lax.shift_left(d32, 16), jnp.float32
        )
        dhi = jax.lax.bitcast_convert_type(
            d32 & jnp.int32(-65536), jnp.float32
        )
        hdn = jnp.maximum(
            jnp.dot(dlo, w1lo_ref[...], preferred_element_type=jnp.float32)
            + jnp.dot(dhi, w1hi_ref[...], preferred_element_type=jnp.float32)
            + b1_ref[...],
            0.0,
        )
        lg = jnp.sum(hdn * w2_ref[...], axis=1, keepdims=True) + b2_ref[...]
        t = t_ref[...]
        row = i * BM + lax.broadcasted_iota(jnp.int32, (BM, 1), 0)
        sidx = (row >= PPAD).astype(jnp.int32) + (row >= 2 * PPAD).astype(
            jnp.int32
        )
        wgt = jnp.where(row - sidx * PPAD < P, SCALE, 0.0)
        bce = jnp.maximum(lg, 0.0) - lg * t + jnp.log1p(jnp.exp(-jnp.abs(lg)))
        part = jnp.reshape(jnp.sum(bce * wgt), (1, 1))

        @pl.when(i == 0)
        def _():
            o_ref[...] = part

        @pl.when(i > 0)
        def _():
            o_ref[...] += part

    return pl.pallas_call(
        body,
        grid=(G,),
        in_specs=[
            pl.BlockSpec((BM, W32), lambda i: (i, 0)),
            pl.BlockSpec((W32, DHID), lambda i: (0, 0)),
            pl.BlockSpec((W32, DHID), lambda i: (0, 0)),
            pl.BlockSpec((1, DHID), lambda i: (0, 0)),
            pl.BlockSpec((1, DHID), lambda i: (0, 0)),
            pl.BlockSpec((1, 1), lambda i: (0, 0)),
            pl.BlockSpec((BM, 1), lambda i: (i, 0)),
        ],
        out_specs=pl.BlockSpec((1, 1), lambda i: (0, 0)),
        out_shape=jax.ShapeDtypeStruct((1, 1), jnp.float32),
    )(dmat, W1lo, W1hi, b1p, w2p, b2, tvec)


def kernel(vertex_features, edge_index, pairs_cells, pairs_cols, pairs_rows,
           targets_cells, targets_cols, targets_rows,
           W_gcnn, b_gcnn, W_h1, b_h1, W_h2, b_h2):
    f32 = jnp.float32
    src = edge_index[0]
    dst = edge_index[1]
    ecomb = jnp.stack(
        [src.reshape(NS, NCE, KE), dst.reshape(NS, NCE, KE)], axis=2
    ).reshape(NS * NCE * 2, KE)

    Wp = jnp.pad(W_gcnn, ((0, 0), (0, DP - 100)))
    bp = jnp.concatenate(
        [b_gcnn, jnp.ones((1,), f32), jnp.zeros((DP - 101,), f32)]
    ).reshape(1, DP)
    W1lo = jnp.pad(W_h1[0:64], ((0, 0), (0, 14)))
    W1hi = jnp.pad(W_h1[64:100], ((0, 28), (0, 14)))
    b1p = jnp.pad(b_h1, (0, 14)).reshape(1, 64)
    w2p = jnp.pad(W_h2[:, 0], (0, 14)).reshape(1, 64)
    b2 = b_h2.reshape(1, 1)

    def padset(x):
        return jnp.pad(x, (0, PPAD - P))

    h2 = _linear(vertex_features, Wp, bp)
    partials = _edge_agg(h2, ecomb)
    gf32 = _finalize_gf(partials)

    pa = jnp.concatenate(
        [padset(pairs_cells[:, 0]), padset(pairs_cols[:, 0]),
         padset(pairs_rows[:, 0])]
    )
    pb = jnp.concatenate(
        [padset(pairs_cells[:, 1]), padset(pairs_cols[:, 1]),
         padset(pairs_rows[:, 1])]
    )
    pcomb = jnp.stack(
        [pa.reshape(NW, NCP, KP), pb.reshape(NW, NCP, KP)], axis=2
    ).reshape(NW * NCP * 2, KP)
    tvec = jnp.concatenate(
        [
            padset(targets_cells.astype(f32)),
            padset(targets_cols.astype(f32)),
            padset(targets_rows.astype(f32)),
        ]
    ).reshape(PTOT, 1)
    dmat32 = _pair_diff(gf32, pcomb)
    return _head_loss(dmat32, W1lo, W1hi, b1p, w2p, b2, tvec)[0, 0]
